# Initial kernel scaffold; baseline (speedup 1.0000x reference)
import jax
import jax.numpy as jnp
from jax.experimental import pallas as pl


def kernel(x, edge_attr, edge_source, edge_target, node_batch, W_ne, b_ne, W_ee, b_ee, Wf, bf, Ws, bs, gamma, beta, Wg, bg, Wr1, br1, Wr2, br2, Wr3, br3, Wr4, br4):
    raise NotImplementedError("write your pallas kernel here")


# speedup vs baseline: 1.2585x; 1.2585x over previous
import functools

import jax
import jax.numpy as jnp
from jax import lax
from jax.experimental import pallas as pl
from jax.experimental.pallas import tpu as pltpu
from jax.experimental.pallas import tpu_sc as plsc

_N = 10000
_E = 320000
_G = 64
_H = 128
_NC, _NS, _L = 2, 16, 16

_IDX_PAD = 647168
_NA = 10240

_HI = lax.Precision.HIGHEST


def _dot(a, b, precision=_HI):
    return lax.dot_general(a, b, (((1,), (0,)), ((), ())),
                           preferred_element_type=jnp.float32,
                           precision=precision)


def _dott(a, b, precision=_HI):
    return lax.dot_general(a, b, (((0,), (0,)), ((), ())),
                           preferred_element_type=jnp.float32,
                           precision=precision)


def _embed_body(x_ref, w_ref, b_ref, o_ref):
    o_ref[...] = jnp.maximum(_dot(x_ref[...], w_ref[...]) + b_ref[...], 0.0)


def _tc_embed(x, w, b):
    return pl.pallas_call(
        _embed_body,
        out_shape=jax.ShapeDtypeStruct((_N, _H), jnp.float32),
    )(x, w, b)


def _msg_body(gs_ref, gt_ref, ea_ref, wee_ref, bee_ref, wf_ref, bf_ref,
              ws_ref, bs_ref, o_ref):
    e = jnp.maximum(_dot(ea_ref[...], wee_ref[...]) + bee_ref[...], 0.0)
    gs = gs_ref[...]
    gt = gt_ref[...]
    wf = wf_ref[...]
    ws = ws_ref[...]
    zf = (_dot(gs, wf[:_H]) + _dot(gt, wf[_H:2 * _H]) + _dot(e, wf[2 * _H:])
          + bf_ref[...])
    zs = (_dot(gs, ws[:_H]) + _dot(gt, ws[_H:2 * _H]) + _dot(e, ws[2 * _H:])
          + bs_ref[...])
    o_ref[...] = jax.nn.sigmoid(zf) * jax.nn.softplus(zs)


def _tc_msg(gathered, edge_attr, wee, bee, wf_i, bf_i, ws_i, bs_i):
    B = 2000
    nblk = _E // B
    full = lambda s: pl.BlockSpec(s, lambda i: (0, 0))
    return pl.pallas_call(
        _msg_body,
        grid=(nblk,),
        in_specs=[
            pl.BlockSpec((B, _H), lambda i: (i, 0)),
            pl.BlockSpec((B, _H), lambda i: (i + nblk, 0)),
            pl.BlockSpec((B, edge_attr.shape[1]), lambda i: (i, 0)),
            full(wee.shape), full(bee.shape), full(wf_i.shape),
            full(bf_i.shape), full(ws_i.shape), full(bs_i.shape),
        ],
        out_specs=pl.BlockSpec((B, _H), lambda i: (i, 0)),
        out_shape=jax.ShapeDtypeStruct((_E, _H), jnp.float32),
    )(gathered, gathered, edge_attr, wee, bee, wf_i, bf_i, ws_i, bs_i)


def _node_body(p_ref, h_ref, gam_ref, bet_ref, nb_ref, hn_ref, g_ref):
    m = (p_ref[0] + p_ref[1])[:_N]
    mean = jnp.mean(m, axis=0, keepdims=True)
    var = jnp.mean(jnp.square(m - mean), axis=0, keepdims=True)
    mn = (m - mean) / jnp.sqrt(var + 1e-5) * gam_ref[...] + bet_ref[...]
    hn = jax.nn.softplus(h_ref[...] + mn)
    hn_ref[...] = hn
    seg = lax.broadcasted_iota(jnp.int32, (_N, _G), 1)
    mask = (nb_ref[...] == seg).astype(jnp.float32)
    g_ref[...] = _dott(mask, hn)


def _tc_node(parts, h, gam, bet, nb2d):
    return pl.pallas_call(
        _node_body,
        out_shape=(jax.ShapeDtypeStruct((_N, _H), jnp.float32),
                   jax.ShapeDtypeStruct((_G, _H), jnp.float32)),
    )(parts, h, gam, bet, nb2d)


def _readout_body(g_ref, wg_ref, bg_ref, w1_ref, b1_ref, w2_ref, b2_ref,
                  w3_ref, b3_ref, w4_ref, b4_ref, o_ref):
    g3 = g_ref[...]
    wg = wg_ref[...]
    bg = bg_ref[...]
    acc = jnp.zeros((_G, _H), jnp.float32)
    for i in range(5):
        acc = acc + _dot(g3[i], wg[i]) + bg[i:i + 1]
    y = jnp.maximum(_dot(acc, w1_ref[...]) + b1_ref[...], 0.0)
    y = jnp.maximum(_dot(y, w2_ref[...]) + b2_ref[...], 0.0)
    y = jnp.maximum(_dot(y, w3_ref[...]) + b3_ref[...], 0.0)
    y = jnp.maximum(_dot(y, w4_ref[...]) + b4_ref[...], 0.0)
    o_ref[...] = y


def _tc_readout(gstack, wg, bg, w1, b1, w2, b2, w3, b3, w4, b4):
    return pl.pallas_call(
        _readout_body,
        out_shape=jax.ShapeDtypeStruct((_G, 1), jnp.float32),
    )(gstack, wg, bg, w1, b1, w2, b2, w3, b3, w4, b4)


def _sc_gather(h, idx2d):
    mesh = plsc.VectorSubcoreMesh(core_axis_name="c", subcore_axis_name="s")
    n = idx2d.shape[0] * 128
    nchunks = n // 128
    cpw = nchunks // (_NC * _NS)
    ng = cpw // 2

    @functools.partial(
        pl.kernel,
        out_type=jax.ShapeDtypeStruct((nchunks, 128, _H), jnp.float32),
        mesh=mesh,
        scratch_types=[
            pltpu.VMEM((4, 128, _H), jnp.float32),
            pltpu.VMEM((4, 128), jnp.int32),
            pltpu.SemaphoreType.DMA((2,)),
            pltpu.SemaphoreType.DMA((2,)),
        ],
    )
    def k(h_hbm, i_hbm, o_hbm, rbuf, ibuf, sem_i, sem_o):
        c = lax.axis_index("c")
        s = lax.axis_index("s")
        w = c * _NS + s
        base = w * cpw

        pltpu.sync_copy(i_hbm.at[pl.ds(base, 2)], ibuf.at[pl.ds(0, 2)])

        @pl.loop(0, ng)
        def _(g):
            p = lax.rem(g, 2)
            q = 1 - p

            @pl.when(g + 1 < ng)
            def _():
                pltpu.async_copy(i_hbm.at[pl.ds(base + (g + 1) * 2, 2)],
                                 ibuf.at[pl.ds(q * 2, 2)], sem_i.at[q])

            @pl.when(g >= 2)
            def _():
                pltpu.make_async_copy(
                    rbuf.at[pl.ds(p * 2, 2)],
                    o_hbm.at[pl.ds(base + (g - 2) * 2, 2)],
                    sem_o.at[p]).wait()

            @pl.when(g >= 1)
            def _():
                pltpu.make_async_copy(
                    i_hbm.at[pl.ds(base + g * 2, 2)],
                    ibuf.at[pl.ds(p * 2, 2)], sem_i.at[p]).wait()

            d0 = pltpu.async_copy(h_hbm.at[ibuf.at[p * 2]],
                                  rbuf.at[p * 2], sem_i.at[p])
            d1 = pltpu.async_copy(h_hbm.at[ibuf.at[p * 2 + 1]],
                                  rbuf.at[p * 2 + 1], sem_i.at[p])
            d0.wait()
            d1.wait()
            pltpu.async_copy(rbuf.at[pl.ds(p * 2, 2)],
                             o_hbm.at[pl.ds(base + g * 2, 2)],
                             sem_o.at[p])

        for gl in (ng - 2, ng - 1):
            pltpu.make_async_copy(
                rbuf.at[pl.ds((gl % 2) * 2, 2)],
                o_hbm.at[pl.ds(base + gl * 2, 2)],
                sem_o.at[gl % 2]).wait()

    return k(h, idx2d).reshape(n, _H)


def _sc_scatter(msg, src2d):
    mesh = plsc.VectorSubcoreMesh(core_axis_name="c", subcore_axis_name="s")
    nsc = _E // 256
    rps = _NA // _NS

    @functools.partial(
        pl.kernel,
        out_type=jax.ShapeDtypeStruct((_NC, _NA, _H), jnp.float32),
        mesh=mesh,
        scratch_types=[
            pltpu.VMEM_SHARED((_NA, _H), jnp.float32),
            pltpu.VMEM((256, _H), jnp.float32),
            pltpu.VMEM((2, 128), jnp.int32),
        ],
    )
    def k(m_hbm, i_hbm, o_hbm, acc, mbuf, ibuf):
        c = lax.axis_index("c")
        s = lax.axis_index("s")
        w = c * _NS + s

        @pl.loop(0, 128)
        def _(r):
            @pl.loop(0, _H, step=_L)
            def _(cc):
                mbuf[r, pl.ds(cc, _L)] = jnp.zeros((_L,), jnp.float32)

        @pl.loop(0, rps // 128)
        def _(j):
            pltpu.sync_copy(mbuf.at[pl.ds(0, 128)],
                            acc.at[pl.ds(s * rps + j * 128, 128)])

        plsc.subcore_barrier()

        @pl.loop(0, (nsc + 31) // 32)
        def _(kk):
            sc = w + kk * (_NC * _NS)

            @pl.when(sc < nsc)
            def _():
                pltpu.sync_copy(m_hbm.at[pl.ds(sc * 256, 256)], mbuf)
                pltpu.sync_copy(i_hbm.at[pl.ds(sc * 2, 2)], ibuf)
                for j in range(2):
                    pltpu.sync_copy(mbuf.at[pl.ds(j * 128, 128)],
                                    acc.at[ibuf.at[j]], add=True)

        plsc.subcore_barrier()
        pltpu.sync_copy(acc.at[pl.ds(s * rps, rps)],
                        o_hbm.at[c, pl.ds(s * rps, rps)])

    return k(msg, src2d)


def kernel(x, edge_attr, edge_source, edge_target, node_batch,
           W_ne, b_ne, W_ee, b_ee, Wf, bf, Ws, bs, gamma, beta,
           Wg, bg, Wr1, br1, Wr2, br2, Wr3, br3, Wr4, br4):
    es = edge_source.astype(jnp.int32)
    et = edge_target.astype(jnp.int32)
    idx_cat = jnp.pad(jnp.concatenate([es, et], 0), (0, _IDX_PAD - 2 * _E))
    idx2d = idx_cat.reshape(_IDX_PAD // 128, 128)
    src2d = es.reshape(_E // 128, 128)
    nb2d = node_batch.astype(jnp.int32).reshape(_N, 1)

    h = _tc_embed(x, W_ne, b_ne.reshape(1, _H))
    glist = []
    for i in range(5):
        gathered = _sc_gather(h, idx2d)
        msg = _tc_msg(gathered, edge_attr, W_ee, b_ee.reshape(1, _H),
                      Wf[i], bf[i].reshape(1, _H), Ws[i], bs[i].reshape(1, _H))
        parts = _sc_scatter(msg, src2d)
        h, g = _tc_node(parts, h, gamma[i].reshape(1, _H),
                        beta[i].reshape(1, _H), nb2d)
        glist.append(g)
    gstack = jnp.stack(glist)
    return _tc_readout(gstack, Wg, bg, Wr1, br1.reshape(1, -1),
                       Wr2, br2.reshape(1, -1), Wr3, br3.reshape(1, -1),
                       Wr4, br4.reshape(1, -1))


# speedup vs baseline: 1.8083x; 1.4369x over previous
import functools

import jax
import jax.numpy as jnp
from jax import lax
from jax.experimental import pallas as pl
from jax.experimental.pallas import tpu as pltpu
from jax.experimental.pallas import tpu_sc as plsc

_N = 10000
_E = 320000
_G = 64
_H = 128
_NC, _NS, _L = 2, 16, 16

_IDX_PAD = 647168
_NA = 10240

_HI = lax.Precision.HIGHEST


def _dot(a, b, precision=_HI):
    return lax.dot_general(a, b, (((1,), (0,)), ((), ())),
                           preferred_element_type=jnp.float32,
                           precision=precision)


def _dott(a, b, precision=_HI):
    return lax.dot_general(a, b, (((0,), (0,)), ((), ())),
                           preferred_element_type=jnp.float32,
                           precision=precision)


def _embed_body(x_ref, w_ref, b_ref, o_ref):
    o_ref[...] = jnp.maximum(_dot(x_ref[...], w_ref[...]) + b_ref[...], 0.0)


def _tc_embed(x, w, b):
    return pl.pallas_call(
        _embed_body,
        out_shape=jax.ShapeDtypeStruct((_N, _H), jnp.float32),
    )(x, w, b)


def _msg_body(gs_ref, gt_ref, ea_ref, wee_ref, bee_ref, wf_ref, bf_ref,
              ws_ref, bs_ref, o_ref):
    bf16 = jnp.bfloat16
    e = jnp.maximum(_dot(ea_ref[...], wee_ref[...]) + bee_ref[...], 0.0)
    e = e.astype(bf16)
    gs = gs_ref[...].astype(bf16)
    gt = gt_ref[...].astype(bf16)
    wf = wf_ref[...].astype(bf16)
    ws = ws_ref[...].astype(bf16)
    d = lambda a, b: _dot(a, b, precision=lax.Precision.DEFAULT)

    def gate(w):
        return d(gs, w[:_H]) + d(gt, w[_H:2 * _H]) + d(e, w[2 * _H:])

    zf = gate(wf) + bf_ref[...]
    zs = gate(ws) + bs_ref[...]
    o_ref[...] = jax.nn.sigmoid(zf) * jax.nn.softplus(zs)


def _tc_msg(gathered, edge_attr, wee, bee, wf_i, bf_i, ws_i, bs_i):
    B = 2000
    nblk = _E // B
    full = lambda s: pl.BlockSpec(s, lambda i: (0, 0))
    return pl.pallas_call(
        _msg_body,
        grid=(nblk,),
        in_specs=[
            pl.BlockSpec((B, _H), lambda i: (i, 0)),
            pl.BlockSpec((B, _H), lambda i: (i + nblk, 0)),
            pl.BlockSpec((B, edge_attr.shape[1]), lambda i: (i, 0)),
            full(wee.shape), full(bee.shape), full(wf_i.shape),
            full(bf_i.shape), full(ws_i.shape), full(bs_i.shape),
        ],
        out_specs=pl.BlockSpec((B, _H), lambda i: (i, 0)),
        out_shape=jax.ShapeDtypeStruct((_E, _H), jnp.float32),
    )(gathered, gathered, edge_attr, wee, bee, wf_i, bf_i, ws_i, bs_i)


def _node_body(p_ref, h_ref, gam_ref, bet_ref, nb_ref, hn_ref, g_ref):
    m = (p_ref[0] + p_ref[1])[:_N]
    mean = jnp.mean(m, axis=0, keepdims=True)
    var = jnp.mean(jnp.square(m - mean), axis=0, keepdims=True)
    mn = (m - mean) / jnp.sqrt(var + 1e-5) * gam_ref[...] + bet_ref[...]
    hn = jax.nn.softplus(h_ref[...] + mn)
    hn_ref[...] = hn
    seg = lax.broadcasted_iota(jnp.int32, (_N, _G), 1)
    mask = (nb_ref[...] == seg).astype(jnp.float32)
    g_ref[...] = _dott(mask, hn)


def _tc_node(parts, h, gam, bet, nb2d):
    return pl.pallas_call(
        _node_body,
        out_shape=(jax.ShapeDtypeStruct((_N, _H), jnp.float32),
                   jax.ShapeDtypeStruct((_G, _H), jnp.float32)),
    )(parts, h, gam, bet, nb2d)


def _readout_body(g_ref, wg_ref, bg_ref, w1_ref, b1_ref, w2_ref, b2_ref,
                  w3_ref, b3_ref, w4_ref, b4_ref, o_ref):
    g3 = g_ref[...]
    wg = wg_ref[...]
    bg = bg_ref[...]
    acc = jnp.zeros((_G, _H), jnp.float32)
    for i in range(5):
        acc = acc + _dot(g3[i], wg[i]) + bg[i:i + 1]
    y = jnp.maximum(_dot(acc, w1_ref[...]) + b1_ref[...], 0.0)
    y = jnp.maximum(_dot(y, w2_ref[...]) + b2_ref[...], 0.0)
    y = jnp.maximum(_dot(y, w3_ref[...]) + b3_ref[...], 0.0)
    y = jnp.maximum(_dot(y, w4_ref[...]) + b4_ref[...], 0.0)
    o_ref[...] = y


def _tc_readout(gstack, wg, bg, w1, b1, w2, b2, w3, b3, w4, b4):
    return pl.pallas_call(
        _readout_body,
        out_shape=jax.ShapeDtypeStruct((_G, 1), jnp.float32),
    )(gstack, wg, bg, w1, b1, w2, b2, w3, b3, w4, b4)


def _sc_gather(h, idx2d):
    mesh = plsc.VectorSubcoreMesh(core_axis_name="c", subcore_axis_name="s")
    n = idx2d.shape[0] * 128
    nchunks = n // 128
    cpw = nchunks // (_NC * _NS)
    ng = cpw // 2

    @functools.partial(
        pl.kernel,
        out_type=jax.ShapeDtypeStruct((nchunks, 128, _H), jnp.float32),
        mesh=mesh,
        scratch_types=[
            pltpu.VMEM((4, 128, _H), jnp.float32),
            pltpu.VMEM((4, 128), jnp.int32),
            pltpu.SemaphoreType.DMA((2,)),
            pltpu.SemaphoreType.DMA((2,)),
        ],
    )
    def k(h_hbm, i_hbm, o_hbm, rbuf, ibuf, sem_i, sem_o):
        c = lax.axis_index("c")
        s = lax.axis_index("s")
        w = c * _NS + s
        base = w * cpw

        pltpu.sync_copy(i_hbm.at[pl.ds(base, 2)], ibuf.at[pl.ds(0, 2)])

        @pl.loop(0, ng)
        def _(g):
            p = lax.rem(g, 2)
            q = 1 - p

            @pl.when(g + 1 < ng)
            def _():
                pltpu.async_copy(i_hbm.at[pl.ds(base + (g + 1) * 2, 2)],
                                 ibuf.at[pl.ds(q * 2, 2)], sem_i.at[q])

            @pl.when(g >= 2)
            def _():
                pltpu.make_async_copy(
                    rbuf.at[pl.ds(p * 2, 2)],
                    o_hbm.at[pl.ds(base + (g - 2) * 2, 2)],
                    sem_o.at[p]).wait()

            @pl.when(g >= 1)
            def _():
                pltpu.make_async_copy(
                    i_hbm.at[pl.ds(base + g * 2, 2)],
                    ibuf.at[pl.ds(p * 2, 2)], sem_i.at[p]).wait()

            d0 = pltpu.async_copy(h_hbm.at[ibuf.at[p * 2]],
                                  rbuf.at[p * 2], sem_i.at[p])
            d1 = pltpu.async_copy(h_hbm.at[ibuf.at[p * 2 + 1]],
                                  rbuf.at[p * 2 + 1], sem_i.at[p])
            d0.wait()
            d1.wait()
            pltpu.async_copy(rbuf.at[pl.ds(p * 2, 2)],
                             o_hbm.at[pl.ds(base + g * 2, 2)],
                             sem_o.at[p])

        for gl in (ng - 2, ng - 1):
            pltpu.make_async_copy(
                rbuf.at[pl.ds((gl % 2) * 2, 2)],
                o_hbm.at[pl.ds(base + gl * 2, 2)],
                sem_o.at[gl % 2]).wait()

    return k(h, idx2d).reshape(n, _H)


def _sc_scatter(msg, src2d):
    mesh = plsc.VectorSubcoreMesh(core_axis_name="c", subcore_axis_name="s")
    nsc = _E // 256
    rps = _NA // _NS

    @functools.partial(
        pl.kernel,
        out_type=jax.ShapeDtypeStruct((_NC, _NA, _H), jnp.float32),
        mesh=mesh,
        scratch_types=[
            pltpu.VMEM_SHARED((_NA, _H), jnp.float32),
            pltpu.VMEM((256, _H), jnp.float32),
            pltpu.VMEM((2, 128), jnp.int32),
        ],
    )
    def k(m_hbm, i_hbm, o_hbm, acc, mbuf, ibuf):
        c = lax.axis_index("c")
        s = lax.axis_index("s")
        w = c * _NS + s

        @pl.loop(0, 128)
        def _(r):
            @pl.loop(0, _H, step=_L)
            def _(cc):
                mbuf[r, pl.ds(cc, _L)] = jnp.zeros((_L,), jnp.float32)

        @pl.loop(0, rps // 128)
        def _(j):
            pltpu.sync_copy(mbuf.at[pl.ds(0, 128)],
                            acc.at[pl.ds(s * rps + j * 128, 128)])

        plsc.subcore_barrier()

        @pl.loop(0, (nsc + 31) // 32)
        def _(kk):
            sc = w + kk * (_NC * _NS)

            @pl.when(sc < nsc)
            def _():
                pltpu.sync_copy(m_hbm.at[pl.ds(sc * 256, 256)], mbuf)
                pltpu.sync_copy(i_hbm.at[pl.ds(sc * 2, 2)], ibuf)
                for j in range(2):
                    pltpu.sync_copy(mbuf.at[pl.ds(j * 128, 128)],
                                    acc.at[ibuf.at[j]], add=True)

        plsc.subcore_barrier()
        pltpu.sync_copy(acc.at[pl.ds(s * rps, rps)],
                        o_hbm.at[c, pl.ds(s * rps, rps)])

    return k(msg, src2d)


def kernel(x, edge_attr, edge_source, edge_target, node_batch,
           W_ne, b_ne, W_ee, b_ee, Wf, bf, Ws, bs, gamma, beta,
           Wg, bg, Wr1, br1, Wr2, br2, Wr3, br3, Wr4, br4):
    es = edge_source.astype(jnp.int32)
    et = edge_target.astype(jnp.int32)
    idx_cat = jnp.pad(jnp.concatenate([es, et], 0), (0, _IDX_PAD - 2 * _E))
    idx2d = idx_cat.reshape(_IDX_PAD // 128, 128)
    src2d = es.reshape(_E // 128, 128)
    nb2d = node_batch.astype(jnp.int32).reshape(_N, 1)

    h = _tc_embed(x, W_ne, b_ne.reshape(1, _H))
    glist = []
    for i in range(5):
        gathered = _sc_gather(h, idx2d)
        msg = _tc_msg(gathered, edge_attr, W_ee, b_ee.reshape(1, _H),
                      Wf[i], bf[i].reshape(1, _H), Ws[i], bs[i].reshape(1, _H))
        parts = _sc_scatter(msg, src2d)
        h, g = _tc_node(parts, h, gamma[i].reshape(1, _H),
                        beta[i].reshape(1, _H), nb2d)
        glist.append(g)
    gstack = jnp.stack(glist)
    return _tc_readout(gstack, Wg, bg, Wr1, br1.reshape(1, -1),
                       Wr2, br2.reshape(1, -1), Wr3, br3.reshape(1, -1),
                       Wr4, br4.reshape(1, -1))


# speedup vs baseline: 2.0533x; 1.1354x over previous
import functools

import jax
import jax.numpy as jnp
from jax import lax
from jax.experimental import pallas as pl
from jax.experimental.pallas import tpu as pltpu
from jax.experimental.pallas import tpu_sc as plsc

_N = 10000
_E = 320000
_G = 64
_H = 128
_NC, _NS, _L = 2, 16, 16

_IDX_PAD = 647168
_NA = 10240

def _dot(a, b):
    return lax.dot_general(a, b, (((1,), (0,)), ((), ())),
                           preferred_element_type=jnp.float32)


def _dott(a, b):
    return lax.dot_general(a, b, (((0,), (0,)), ((), ())),
                           preferred_element_type=jnp.float32)


def _split(a):
    ah = a.astype(jnp.bfloat16)
    return ah, (a - ah.astype(jnp.float32)).astype(jnp.bfloat16)


def _dot3(a, b):
    ah, al = _split(a)
    bh, bl = _split(b)
    return _dot(ah, bh) + (_dot(ah, bl) + _dot(al, bh))


def _embed_body(x_ref, w_ref, b_ref, o_ref):
    o_ref[...] = jnp.maximum(_dot(x_ref[...], w_ref[...]) + b_ref[...], 0.0)


def _tc_embed(x, w, b):
    return pl.pallas_call(
        _embed_body,
        out_shape=jax.ShapeDtypeStruct((_N, _H), jnp.float32),
    )(x, w, b)


def _msg_body(gs_ref, gt_ref, ea_ref, wee_ref, bee_ref, wf_ref, bf_ref,
              ws_ref, bs_ref, o_ref):
    e = jnp.maximum(_dot(ea_ref[...], wee_ref[...]) + bee_ref[...], 0.0)
    gs = gs_ref[...]
    gt = gt_ref[...]
    wf = wf_ref[...]
    ws = ws_ref[...]

    def gate(w):
        return (_dot(gs, w[:_H]) + _dot(gt, w[_H:2 * _H])
                + _dot(e, w[2 * _H:]))

    zf = gate(wf) + bf_ref[...]
    zs = gate(ws) + bs_ref[...]
    o_ref[...] = jax.nn.sigmoid(zf) * jax.nn.softplus(zs)


def _tc_msg(gathered, edge_attr, wee, bee, wf_i, bf_i, ws_i, bs_i):
    B = 2000
    nblk = _E // B
    full = lambda s: pl.BlockSpec(s, lambda i: (0, 0))
    return pl.pallas_call(
        _msg_body,
        grid=(nblk,),
        in_specs=[
            pl.BlockSpec((B, _H), lambda i: (i, 0)),
            pl.BlockSpec((B, _H), lambda i: (i + nblk, 0)),
            pl.BlockSpec((B, edge_attr.shape[1]), lambda i: (i, 0)),
            full(wee.shape), full(bee.shape), full(wf_i.shape),
            full(bf_i.shape), full(ws_i.shape), full(bs_i.shape),
        ],
        out_specs=pl.BlockSpec((B, _H), lambda i: (i, 0)),
        out_shape=jax.ShapeDtypeStruct((_E, _H), jnp.float32),
    )(gathered, gathered, edge_attr, wee, bee, wf_i, bf_i, ws_i, bs_i)


def _node_body(p_ref, h_ref, gam_ref, bet_ref, nb_ref, hn_ref, g_ref):
    m = (p_ref[0] + p_ref[1])[:_N]
    mean = jnp.mean(m, axis=0, keepdims=True)
    var = jnp.mean(jnp.square(m - mean), axis=0, keepdims=True)
    mn = (m - mean) / jnp.sqrt(var + 1e-5) * gam_ref[...] + bet_ref[...]
    hn = jax.nn.softplus(h_ref[...] + mn)
    hn_ref[...] = hn
    seg = lax.broadcasted_iota(jnp.int32, (_N, _G), 1)
    mask = (nb_ref[...] == seg).astype(jnp.bfloat16)
    hh, hl = _split(hn)
    g_ref[...] = _dott(mask, hh) + _dott(mask, hl)


def _tc_node(parts, h, gam, bet, nb2d):
    return pl.pallas_call(
        _node_body,
        out_shape=(jax.ShapeDtypeStruct((_N, _H), jnp.float32),
                   jax.ShapeDtypeStruct((_G, _H), jnp.float32)),
    )(parts, h, gam, bet, nb2d)


def _readout_body(g_ref, wg_ref, bg_ref, w1_ref, b1_ref, w2_ref, b2_ref,
                  w3_ref, b3_ref, w4_ref, b4_ref, o_ref):
    g3 = g_ref[...]
    wg = wg_ref[...]
    bg = bg_ref[...]
    acc = jnp.zeros((_G, _H), jnp.float32)
    for i in range(5):
        acc = acc + _dot(g3[i], wg[i]) + bg[i:i + 1]
    y = jnp.maximum(_dot(acc, w1_ref[...]) + b1_ref[...], 0.0)
    y = jnp.maximum(_dot(y, w2_ref[...]) + b2_ref[...], 0.0)
    y = jnp.maximum(_dot(y, w3_ref[...]) + b3_ref[...], 0.0)
    y = jnp.maximum(_dot(y, w4_ref[...]) + b4_ref[...], 0.0)
    o_ref[...] = y


def _tc_readout(gstack, wg, bg, w1, b1, w2, b2, w3, b3, w4, b4):
    return pl.pallas_call(
        _readout_body,
        out_shape=jax.ShapeDtypeStruct((_G, 1), jnp.float32),
    )(gstack, wg, bg, w1, b1, w2, b2, w3, b3, w4, b4)


def _sc_gather(h, idx2d):
    mesh = plsc.VectorSubcoreMesh(core_axis_name="c", subcore_axis_name="s")
    n = idx2d.shape[0] * 128
    nchunks = n // 128
    cpw = nchunks // (_NC * _NS)
    ng = cpw // 2

    @functools.partial(
        pl.kernel,
        out_type=jax.ShapeDtypeStruct((nchunks, 128, _H), jnp.float32),
        mesh=mesh,
        scratch_types=[
            pltpu.VMEM((4, 128, _H), jnp.float32),
            pltpu.VMEM((6, 128), jnp.int32),
            pltpu.SemaphoreType.DMA((3,)),
            pltpu.SemaphoreType.DMA((2,)),
            pltpu.SemaphoreType.DMA((2,)),
        ],
    )
    def k(h_hbm, i_hbm, o_hbm, rbuf, ibuf, sem_i, sem_g, sem_o):
        c = lax.axis_index("c")
        s = lax.axis_index("s")
        w = c * _NS + s
        base = w * cpw

        pltpu.async_copy(i_hbm.at[pl.ds(base, 2)],
                         ibuf.at[pl.ds(0, 2)], sem_i.at[0])
        pltpu.async_copy(i_hbm.at[pl.ds(base + 2, 2)],
                         ibuf.at[pl.ds(2, 2)], sem_i.at[1])

        @pl.loop(0, ng)
        def _(g):
            p = lax.rem(g, 2)
            q = 1 - p
            r = lax.rem(g, 3)

            @pl.when(g >= 2)
            def _():
                pltpu.make_async_copy(
                    rbuf.at[pl.ds(p * 2, 2)],
                    o_hbm.at[pl.ds(base + (g - 2) * 2, 2)],
                    sem_o.at[p]).wait()

            pltpu.make_async_copy(
                i_hbm.at[pl.ds(base + g * 2, 2)],
                ibuf.at[pl.ds(r * 2, 2)], sem_i.at[r]).wait()

            pltpu.async_copy(h_hbm.at[ibuf.at[r * 2]],
                             rbuf.at[p * 2], sem_g.at[p])
            pltpu.async_copy(h_hbm.at[ibuf.at[r * 2 + 1]],
                             rbuf.at[p * 2 + 1], sem_g.at[p])

            @pl.when(g >= 1)
            def _():
                rq = lax.rem(g + 2, 3)
                for j in range(2):
                    pltpu.make_async_copy(
                        h_hbm.at[ibuf.at[rq * 2 + j]],
                        rbuf.at[q * 2 + j], sem_g.at[q]).wait()
                pltpu.async_copy(rbuf.at[pl.ds(q * 2, 2)],
                                 o_hbm.at[pl.ds(base + (g - 1) * 2, 2)],
                                 sem_o.at[q])

            @pl.when(g + 2 < ng)
            def _():
                rn = lax.rem(g + 2, 3)
                pltpu.async_copy(i_hbm.at[pl.ds(base + (g + 2) * 2, 2)],
                                 ibuf.at[pl.ds(rn * 2, 2)], sem_i.at[rn])

        gl = ng - 1
        pl_ = gl % 2
        rl = gl % 3
        for j in range(2):
            pltpu.make_async_copy(
                h_hbm.at[ibuf.at[rl * 2 + j]],
                rbuf.at[pl_ * 2 + j], sem_g.at[pl_]).wait()
        pltpu.async_copy(rbuf.at[pl.ds(pl_ * 2, 2)],
                         o_hbm.at[pl.ds(base + gl * 2, 2)], sem_o.at[pl_])
        for gq in (ng - 2, ng - 1):
            pltpu.make_async_copy(
                rbuf.at[pl.ds((gq % 2) * 2, 2)],
                o_hbm.at[pl.ds(base + gq * 2, 2)],
                sem_o.at[gq % 2]).wait()

    return k(h, idx2d).reshape(n, _H)


def _sc_scatter(msg, src2d):
    mesh = plsc.VectorSubcoreMesh(core_axis_name="c", subcore_axis_name="s")
    nsc = _E // 256
    rps = _NA // _NS

    @functools.partial(
        pl.kernel,
        out_type=jax.ShapeDtypeStruct((_NC, _NA, _H), jnp.float32),
        mesh=mesh,
        scratch_types=[
            pltpu.VMEM_SHARED((_NA, _H), jnp.float32),
            pltpu.VMEM((256, _H), jnp.float32),
            pltpu.VMEM((2, 128), jnp.int32),
        ],
    )
    def k(m_hbm, i_hbm, o_hbm, acc, mbuf, ibuf):
        c = lax.axis_index("c")
        s = lax.axis_index("s")
        w = c * _NS + s

        @pl.loop(0, 128)
        def _(r):
            @pl.loop(0, _H, step=_L)
            def _(cc):
                mbuf[r, pl.ds(cc, _L)] = jnp.zeros((_L,), jnp.float32)

        @pl.loop(0, rps // 128)
        def _(j):
            pltpu.sync_copy(mbuf.at[pl.ds(0, 128)],
                            acc.at[pl.ds(s * rps + j * 128, 128)])

        plsc.subcore_barrier()

        @pl.loop(0, (nsc + 31) // 32)
        def _(kk):
            sc = w + kk * (_NC * _NS)

            @pl.when(sc < nsc)
            def _():
                pltpu.sync_copy(m_hbm.at[pl.ds(sc * 256, 256)], mbuf)
                pltpu.sync_copy(i_hbm.at[pl.ds(sc * 2, 2)], ibuf)
                for j in range(2):
                    pltpu.sync_copy(mbuf.at[pl.ds(j * 128, 128)],
                                    acc.at[ibuf.at[j]], add=True)

        plsc.subcore_barrier()
        pltpu.sync_copy(acc.at[pl.ds(s * rps, rps)],
                        o_hbm.at[c, pl.ds(s * rps, rps)])

    return k(msg, src2d)


def kernel(x, edge_attr, edge_source, edge_target, node_batch,
           W_ne, b_ne, W_ee, b_ee, Wf, bf, Ws, bs, gamma, beta,
           Wg, bg, Wr1, br1, Wr2, br2, Wr3, br3, Wr4, br4):
    es = edge_source.astype(jnp.int32)
    et = edge_target.astype(jnp.int32)
    idx_cat = jnp.pad(jnp.concatenate([es, et], 0), (0, _IDX_PAD - 2 * _E))
    idx2d = idx_cat.reshape(_IDX_PAD // 128, 128)
    src2d = es.reshape(_E // 128, 128)
    nb2d = node_batch.astype(jnp.int32).reshape(_N, 1)

    h = _tc_embed(x, W_ne, b_ne.reshape(1, _H))
    glist = []
    for i in range(5):
        gathered = _sc_gather(h, idx2d)
        msg = _tc_msg(gathered, edge_attr, W_ee, b_ee.reshape(1, _H),
                      Wf[i], bf[i].reshape(1, _H), Ws[i], bs[i].reshape(1, _H))
        parts = _sc_scatter(msg, src2d)
        h, g = _tc_node(parts, h, gamma[i].reshape(1, _H),
                        beta[i].reshape(1, _H), nb2d)
        glist.append(g)
    gstack = jnp.stack(glist)
    return _tc_readout(gstack, Wg, bg, Wr1, br1.reshape(1, -1),
                       Wr2, br2.reshape(1, -1), Wr3, br3.reshape(1, -1),
                       Wr4, br4.reshape(1, -1))
